# baseline (speedup 1.0000x reference)
import functools

import jax
import jax.numpy as jnp
from jax import lax
from jax.experimental import pallas as pl
from jax.experimental.pallas import tpu as pltpu
from jax.experimental.pallas import tpu_sc as plsc

N = 10000
EDGES = 320000
IN_F = 128
H1F = 64
H2F = 32
NG = 64
NT = 2

NC = 2
NS = 16
NW = NC * NS
LANES = 16

NPAD = 10240
CHUNK = 128
NCHTOT = 2560
EPAD = NCHTOT * CHUNK
RPT = NPAD // NS
ROWBLK = 2048
NROWBLK = NPAD // ROWBLK

PROP_SPLIT = (136, 24)
DEG_SPLIT = (104, 56)

_HIGH = lax.Precision.HIGHEST


def _sc_mesh():
    return plsc.VectorSubcoreMesh(core_axis_name="c", subcore_axis_name="s")


def _deg_body(dst_hbm, out_hbm, dstv, degv):
    cid = lax.axis_index("c")
    sid = lax.axis_index("s")
    wid = sid * NC + cid
    zeros = jnp.zeros((LANES,), jnp.float32)

    def zb(i, carry):
        degv[pl.ds(i * LANES, LANES)] = zeros
        return carry

    lax.fori_loop(0, NPAD // LANES, zb, 0)

    ones = jnp.ones((LANES,), jnp.float32)
    per_chunk = CHUNK // LANES

    def run(base, nch):
        pltpu.sync_copy(dst_hbm.at[pl.ds(base, nch)], dstv.at[pl.ds(0, nch)])

        def eb(i, carry):
            c = i // per_chunk
            k = i % per_chunk
            idx = dstv[c, pl.ds(k * LANES, LANES)]
            plsc.addupdate_scatter(degv, [idx], ones)
            return carry

        lax.fori_loop(0, nch * per_chunk, eb, 0)

    n0, n1 = DEG_SPLIT

    @pl.when(cid == 0)
    def _():
        run(sid * n0, n0)

    @pl.when(cid != 0)
    def _():
        run(NS * n0 + sid * n1, n1)

    pltpu.sync_copy(degv, out_hbm.at[wid])


@jax.jit
def _deg_call(dstp):
    fn = functools.partial(
        pl.kernel,
        out_type=jax.ShapeDtypeStruct((NW, NPAD), jnp.float32),
        mesh=_sc_mesh(),
        scratch_types=[
            pltpu.VMEM((max(DEG_SPLIT), CHUNK), jnp.int32),
            pltpu.VMEM((NPAD,), jnp.float32),
        ],
        compiler_params=pltpu.CompilerParams(
            needs_layout_passes=False, use_tc_tiling_on_sc=False),
    )(_deg_body)
    return fn(dstp)


def _make_prop(F):
    def body(h_hbm, src_hbm, dst_hbm, out_hbm,
             srcv, dstv, buf0, buf1, buf2, buf3, acc,
             gs0, gs1, gs2, gs3, ss0, ss1, ss2, ss3):
        cid = lax.axis_index("c")
        sid = lax.axis_index("s")
        rlo = sid * RPT

        @pl.when(cid == 0)
        def _():
            pltpu.sync_copy(h_hbm.at[pl.ds(rlo, RPT)], acc.at[pl.ds(rlo, RPT)])

        @pl.when(cid != 0)
        def _():
            zeros = jnp.zeros((LANES,), jnp.float32)
            for r in range(CHUNK):
                for c in range(F // LANES):
                    buf0[r, pl.ds(c * LANES, LANES)] = zeros
            for p in range(RPT // CHUNK):
                pltpu.sync_copy(buf0, acc.at[pl.ds(rlo + p * CHUNK, CHUNK)])

        plsc.subcore_barrier()

        bufs = (buf0, buf1, buf2, buf3)
        gsems = (gs0, gs1, gs2, gs3)
        ssems = (ss0, ss1, ss2, ss3)

        def run(base, nch):
            pltpu.sync_copy(src_hbm.at[pl.ds(base, nch)], srcv.at[pl.ds(0, nch)])
            pltpu.sync_copy(dst_hbm.at[pl.ds(base, nch)], dstv.at[pl.ds(0, nch)])

            pltpu.async_copy(h_hbm.at[srcv.at[0]], bufs[0], gsems[0])
            pltpu.async_copy(h_hbm.at[srcv.at[1]], bufs[1], gsems[1])

            def step(k, carry):
                for b in range(4):
                    j = k * 4 + b
                    nb = (b + 2) % 4
                    jm2 = jnp.maximum(j - 2, 0)

                    def drain_and_prefetch():
                        pltpu.make_async_copy(
                            bufs[nb], acc.at[dstv.at[jm2]], ssems[nb]
                        ).wait()
                        pltpu.async_copy(
                            h_hbm.at[srcv.at[j + 2]], bufs[nb], gsems[nb])

                    if b < 2:
                        @pl.when(k > 0)
                        def _():
                            drain_and_prefetch()

                        @pl.when(k == 0)
                        def _():
                            pltpu.async_copy(
                                h_hbm.at[srcv.at[j + 2]], bufs[nb], gsems[nb])
                    else:
                        pltpu.make_async_copy(
                            bufs[nb], acc.at[dstv.at[jm2]], ssems[nb]
                        ).wait()

                        @pl.when(k < nch // 4 - 1)
                        def _():
                            pltpu.async_copy(
                                h_hbm.at[srcv.at[j + 2]], bufs[nb], gsems[nb])

                    pltpu.make_async_copy(
                        h_hbm.at[srcv.at[j]], bufs[b], gsems[b]).wait()
                    pltpu.async_copy(bufs[b], acc.at[dstv.at[j]], ssems[b],
                                     add=True)
                return carry

            lax.fori_loop(0, nch // 4, step, 0)
            pltpu.make_async_copy(
                bufs[2], acc.at[dstv.at[nch - 2]], ssems[2]).wait()
            pltpu.make_async_copy(
                bufs[3], acc.at[dstv.at[nch - 1]], ssems[3]).wait()

        n0, n1 = PROP_SPLIT

        @pl.when(cid == 0)
        def _():
            run(sid * n0, n0)

        @pl.when(cid != 0)
        def _():
            run(NS * n0 + sid * n1, n1)

        plsc.subcore_barrier()
        pltpu.sync_copy(acc.at[pl.ds(rlo, RPT)], out_hbm.at[cid, pl.ds(rlo, RPT)])

    @jax.jit
    def call(h, srcp, dstp):
        fn = functools.partial(
            pl.kernel,
            out_type=jax.ShapeDtypeStruct((NC, NPAD, F), jnp.float32),
            mesh=_sc_mesh(),
            scratch_types=(
                [pltpu.VMEM((max(PROP_SPLIT), CHUNK), jnp.int32)] * 2
                + [pltpu.VMEM((CHUNK, F), jnp.float32)] * 4
                + [pltpu.VMEM_SHARED((NPAD, F), jnp.float32)]
                + [pltpu.SemaphoreType.DMA] * 8
            ),
            compiler_params=pltpu.CompilerParams(
                needs_layout_passes=False, use_tc_tiling_on_sc=False),
        )(body)
        return fn(h, srcp, dstp)

    return call


_prop64 = _make_prop(H1F)
_prop32 = _make_prop(H2F)


def _b_body(degp_ref, x_ref, w1_ref, dis_ref, h1p_ref):
    s = lax.dot_general(degp_ref[...], jnp.ones((NW, 1), jnp.float32),
                        (((0,), (0,)), ((), ())), precision=_HIGH)
    dis = lax.rsqrt(s + 1.0)
    h = lax.dot_general(x_ref[...], w1_ref[...], (((1,), (0,)), ((), ())))
    dis_ref[...] = dis
    h1p_ref[...] = h * dis


@jax.jit
def _b_call(degp, xp, w1):
    return pl.pallas_call(
        _b_body,
        grid=(NROWBLK,),
        in_specs=[
            pl.BlockSpec((NW, ROWBLK), lambda i: (0, i)),
            pl.BlockSpec((ROWBLK, IN_F), lambda i: (i, 0)),
            pl.BlockSpec((IN_F, H1F), lambda i: (0, 0)),
        ],
        out_specs=[
            pl.BlockSpec((ROWBLK, 1), lambda i: (i, 0)),
            pl.BlockSpec((ROWBLK, H1F), lambda i: (i, 0)),
        ],
        out_shape=[
            jax.ShapeDtypeStruct((NPAD, 1), jnp.float32),
            jax.ShapeDtypeStruct((NPAD, H1F), jnp.float32),
        ],
    )(degp, xp, w1)


def _d_body(s_ref, dis_ref, b1_ref, w2_ref, h2p_ref):
    dis = dis_ref[...]
    s = s_ref[...]
    u = (s[0] + s[1]) * dis + b1_ref[...]
    z = jnp.where(u >= 0, u, 0.01 * u)
    h = lax.dot_general(z, w2_ref[...], (((1,), (0,)), ((), ())))
    h2p_ref[...] = h * dis


@jax.jit
def _d_call(s1, dis, b1, w2):
    return pl.pallas_call(
        _d_body,
        grid=(NROWBLK,),
        in_specs=[
            pl.BlockSpec((NC, ROWBLK, H1F), lambda i: (0, i, 0)),
            pl.BlockSpec((ROWBLK, 1), lambda i: (i, 0)),
            pl.BlockSpec((1, H1F), lambda i: (0, 0)),
            pl.BlockSpec((H1F, H2F), lambda i: (0, 0)),
        ],
        out_specs=pl.BlockSpec((ROWBLK, H2F), lambda i: (i, 0)),
        out_shape=jax.ShapeDtypeStruct((NPAD, H2F), jnp.float32),
    )(s1, dis, b1, w2)


def _f_body(s_ref, dis_ref, b2_ref, batch_ref, wlin_ref, blin_ref,
            out_ref, gsum, gcnt):
    i = pl.program_id(0)

    @pl.when(i == 0)
    def _():
        gsum[...] = jnp.zeros_like(gsum)
        gcnt[...] = jnp.zeros_like(gcnt)

    dis = dis_ref[...]
    s = s_ref[...]
    u = (s[0] + s[1]) * dis + b2_ref[...]
    z = jnp.where(u >= 0, u, 0.01 * u)
    gids = lax.broadcasted_iota(jnp.int32, (ROWBLK, NG), 1)
    m = (batch_ref[...] == gids).astype(jnp.float32)
    gsum[...] += lax.dot_general(m, z, (((0,), (0,)), ((), ())),
                                 precision=_HIGH)
    gcnt[...] += lax.dot_general(m, jnp.ones((ROWBLK, 1), jnp.float32),
                                 (((0,), (0,)), ((), ())), precision=_HIGH)

    @pl.when(i == NROWBLK - 1)
    def _():
        g = gsum[...] / jnp.maximum(gcnt[...], 1.0)
        out_ref[...] = lax.dot_general(g, wlin_ref[...],
                                       (((1,), (0,)), ((), ()))) + blin_ref[...]


@jax.jit
def _f_call(s2, dis, b2, batchp, wlin, blin):
    return pl.pallas_call(
        _f_body,
        grid=(NROWBLK,),
        in_specs=[
            pl.BlockSpec((NC, ROWBLK, H2F), lambda i: (0, i, 0)),
            pl.BlockSpec((ROWBLK, 1), lambda i: (i, 0)),
            pl.BlockSpec((1, H2F), lambda i: (0, 0)),
            pl.BlockSpec((ROWBLK, 1), lambda i: (i, 0)),
            pl.BlockSpec((H2F, NT), lambda i: (0, 0)),
            pl.BlockSpec((1, NT), lambda i: (0, 0)),
        ],
        out_specs=pl.BlockSpec((NG, NT), lambda i: (0, 0)),
        out_shape=jax.ShapeDtypeStruct((NG, NT), jnp.float32),
        scratch_shapes=[
            pltpu.VMEM((NG, H2F), jnp.float32),
            pltpu.VMEM((NG, 1), jnp.float32),
        ],
    )(s2, dis, b2, batchp, wlin, blin)


def kernel(x, edge_index, batch, W1, b1, W2, b2, Wlin, blin):
    src = edge_index[0]
    dst = edge_index[1]
    pad = jnp.full((EPAD - EDGES,), N, jnp.int32)
    srcp = jnp.concatenate([src, pad]).reshape(NCHTOT, CHUNK)
    dstp = jnp.concatenate([dst, pad]).reshape(NCHTOT, CHUNK)
    xp = jnp.pad(x, ((0, NPAD - N), (0, 0)))
    batchp = jnp.pad(batch, (0, NPAD - N), constant_values=NG).reshape(NPAD, 1)

    degp = _deg_call(dstp)
    dis, h1p = _b_call(degp, xp, W1)
    s1 = _prop64(h1p, srcp, dstp)
    h2p = _d_call(s1, dis, b1.reshape(1, H1F), W2)
    s2 = _prop32(h2p, srcp, dstp)
    return _f_call(s2, dis, b2.reshape(1, H2F), batchp,
                   Wlin, blin.reshape(1, NT))


# speedup vs baseline: 1.0043x; 1.0043x over previous
import functools

import jax
import jax.numpy as jnp
from jax import lax
from jax.experimental import pallas as pl
from jax.experimental.pallas import tpu as pltpu
from jax.experimental.pallas import tpu_sc as plsc

N = 10000
EDGES = 320000
IN_F = 128
H1F = 64
H2F = 32
NG = 64
NT = 2

NC = 2
NS = 16
NW = NC * NS
LANES = 16

NPAD = 10240
CHUNK = 128
NCHTOT = 2560
EPAD = NCHTOT * CHUNK
RPT = NPAD // NS
ROWBLK = 2048
NROWBLK = NPAD // ROWBLK

PROP_SPLIT = (104, 56)
DEG_SPLIT = (104, 56)

_HIGH = lax.Precision.HIGHEST


def _sc_mesh():
    return plsc.VectorSubcoreMesh(core_axis_name="c", subcore_axis_name="s")


def _deg_body(dst_hbm, out_hbm, dstv, degv):
    cid = lax.axis_index("c")
    sid = lax.axis_index("s")
    wid = sid * NC + cid
    zeros = jnp.zeros((LANES,), jnp.float32)

    def zb(i, carry):
        degv[pl.ds(i * LANES, LANES)] = zeros
        return carry

    lax.fori_loop(0, NPAD // LANES, zb, 0)

    ones = jnp.ones((LANES,), jnp.float32)
    per_chunk = CHUNK // LANES

    def run(base, nch):
        pltpu.sync_copy(dst_hbm.at[pl.ds(base, nch)], dstv.at[pl.ds(0, nch)])

        def eb(i, carry):
            c = i // per_chunk
            k = i % per_chunk
            idx = dstv[c, pl.ds(k * LANES, LANES)]
            plsc.addupdate_scatter(degv, [idx], ones)
            return carry

        lax.fori_loop(0, nch * per_chunk, eb, 0)

    n0, n1 = DEG_SPLIT

    @pl.when(cid == 0)
    def _():
        run(sid * n0, n0)

    @pl.when(cid != 0)
    def _():
        run(NS * n0 + sid * n1, n1)

    pltpu.sync_copy(degv, out_hbm.at[wid])


@jax.jit
def _deg_call(dstp):
    fn = functools.partial(
        pl.kernel,
        out_type=jax.ShapeDtypeStruct((NW, NPAD), jnp.float32),
        mesh=_sc_mesh(),
        scratch_types=[
            pltpu.VMEM((max(DEG_SPLIT), CHUNK), jnp.int32),
            pltpu.VMEM((NPAD,), jnp.float32),
        ],
        compiler_params=pltpu.CompilerParams(
            needs_layout_passes=False, use_tc_tiling_on_sc=False),
    )(_deg_body)
    return fn(dstp)


def _make_prop(F):
    def body(h_hbm, src_hbm, dst_hbm, out_hbm,
             srcv, dstv, buf0, buf1, buf2, buf3, acc,
             gs0, gs1, gs2, gs3, ss0, ss1, ss2, ss3):
        cid = lax.axis_index("c")
        sid = lax.axis_index("s")
        rlo = sid * RPT

        @pl.when(cid == 0)
        def _():
            pltpu.sync_copy(h_hbm.at[pl.ds(rlo, RPT)], acc.at[pl.ds(rlo, RPT)])

        @pl.when(cid != 0)
        def _():
            zeros = jnp.zeros((LANES,), jnp.float32)
            for r in range(CHUNK):
                for c in range(F // LANES):
                    buf0[r, pl.ds(c * LANES, LANES)] = zeros
            for p in range(RPT // CHUNK):
                pltpu.sync_copy(buf0, acc.at[pl.ds(rlo + p * CHUNK, CHUNK)])

        plsc.subcore_barrier()

        bufs = (buf0, buf1, buf2, buf3)
        gsems = (gs0, gs1, gs2, gs3)
        ssems = (ss0, ss1, ss2, ss3)

        def run(base, nch):
            pltpu.sync_copy(src_hbm.at[pl.ds(base, nch)], srcv.at[pl.ds(0, nch)])
            pltpu.sync_copy(dst_hbm.at[pl.ds(base, nch)], dstv.at[pl.ds(0, nch)])

            pltpu.async_copy(h_hbm.at[srcv.at[0]], bufs[0], gsems[0])
            pltpu.async_copy(h_hbm.at[srcv.at[1]], bufs[1], gsems[1])

            def step(k, carry):
                for b in range(4):
                    j = k * 4 + b
                    nb = (b + 2) % 4
                    jm2 = jnp.maximum(j - 2, 0)

                    def drain_and_prefetch():
                        pltpu.make_async_copy(
                            bufs[nb], acc.at[dstv.at[jm2]], ssems[nb]
                        ).wait()
                        pltpu.async_copy(
                            h_hbm.at[srcv.at[j + 2]], bufs[nb], gsems[nb])

                    if b < 2:
                        @pl.when(k > 0)
                        def _():
                            drain_and_prefetch()

                        @pl.when(k == 0)
                        def _():
                            pltpu.async_copy(
                                h_hbm.at[srcv.at[j + 2]], bufs[nb], gsems[nb])
                    else:
                        pltpu.make_async_copy(
                            bufs[nb], acc.at[dstv.at[jm2]], ssems[nb]
                        ).wait()

                        @pl.when(k < nch // 4 - 1)
                        def _():
                            pltpu.async_copy(
                                h_hbm.at[srcv.at[j + 2]], bufs[nb], gsems[nb])

                    pltpu.make_async_copy(
                        h_hbm.at[srcv.at[j]], bufs[b], gsems[b]).wait()
                    pltpu.async_copy(bufs[b], acc.at[dstv.at[j]], ssems[b],
                                     add=True)
                return carry

            lax.fori_loop(0, nch // 4, step, 0)
            pltpu.make_async_copy(
                bufs[2], acc.at[dstv.at[nch - 2]], ssems[2]).wait()
            pltpu.make_async_copy(
                bufs[3], acc.at[dstv.at[nch - 1]], ssems[3]).wait()

        n0, n1 = PROP_SPLIT

        @pl.when(cid == 0)
        def _():
            run(sid * n0, n0)

        @pl.when(cid != 0)
        def _():
            run(NS * n0 + sid * n1, n1)

        plsc.subcore_barrier()
        pltpu.sync_copy(acc.at[pl.ds(rlo, RPT)], out_hbm.at[cid, pl.ds(rlo, RPT)])

    @jax.jit
    def call(h, srcp, dstp):
        fn = functools.partial(
            pl.kernel,
            out_type=jax.ShapeDtypeStruct((NC, NPAD, F), jnp.float32),
            mesh=_sc_mesh(),
            scratch_types=(
                [pltpu.VMEM((max(PROP_SPLIT), CHUNK), jnp.int32)] * 2
                + [pltpu.VMEM((CHUNK, F), jnp.float32)] * 4
                + [pltpu.VMEM_SHARED((NPAD, F), jnp.float32)]
                + [pltpu.SemaphoreType.DMA] * 8
            ),
            compiler_params=pltpu.CompilerParams(
                needs_layout_passes=False, use_tc_tiling_on_sc=False),
        )(body)
        return fn(h, srcp, dstp)

    return call


_prop64 = _make_prop(H1F)
_prop32 = _make_prop(H2F)


def _b_body(degp_ref, x_ref, w1_ref, dis_ref, h1p_ref):
    s = lax.dot_general(degp_ref[...], jnp.ones((NW, 1), jnp.float32),
                        (((0,), (0,)), ((), ())), precision=_HIGH)
    dis = lax.rsqrt(s + 1.0)
    h = lax.dot_general(x_ref[...], w1_ref[...], (((1,), (0,)), ((), ())))
    dis_ref[...] = dis
    h1p_ref[...] = h * dis


@jax.jit
def _b_call(degp, xp, w1):
    return pl.pallas_call(
        _b_body,
        grid=(NROWBLK,),
        in_specs=[
            pl.BlockSpec((NW, ROWBLK), lambda i: (0, i)),
            pl.BlockSpec((ROWBLK, IN_F), lambda i: (i, 0)),
            pl.BlockSpec((IN_F, H1F), lambda i: (0, 0)),
        ],
        out_specs=[
            pl.BlockSpec((ROWBLK, 1), lambda i: (i, 0)),
            pl.BlockSpec((ROWBLK, H1F), lambda i: (i, 0)),
        ],
        out_shape=[
            jax.ShapeDtypeStruct((NPAD, 1), jnp.float32),
            jax.ShapeDtypeStruct((NPAD, H1F), jnp.float32),
        ],
    )(degp, xp, w1)


def _d_body(s_ref, dis_ref, b1_ref, w2_ref, h2p_ref):
    dis = dis_ref[...]
    s = s_ref[...]
    u = (s[0] + s[1]) * dis + b1_ref[...]
    z = jnp.where(u >= 0, u, 0.01 * u)
    h = lax.dot_general(z, w2_ref[...], (((1,), (0,)), ((), ())))
    h2p_ref[...] = h * dis


@jax.jit
def _d_call(s1, dis, b1, w2):
    return pl.pallas_call(
        _d_body,
        grid=(NROWBLK,),
        in_specs=[
            pl.BlockSpec((NC, ROWBLK, H1F), lambda i: (0, i, 0)),
            pl.BlockSpec((ROWBLK, 1), lambda i: (i, 0)),
            pl.BlockSpec((1, H1F), lambda i: (0, 0)),
            pl.BlockSpec((H1F, H2F), lambda i: (0, 0)),
        ],
        out_specs=pl.BlockSpec((ROWBLK, H2F), lambda i: (i, 0)),
        out_shape=jax.ShapeDtypeStruct((NPAD, H2F), jnp.float32),
    )(s1, dis, b1, w2)


def _f_body(s_ref, dis_ref, b2_ref, batch_ref, wlin_ref, blin_ref,
            out_ref, gsum, gcnt):
    i = pl.program_id(0)

    @pl.when(i == 0)
    def _():
        gsum[...] = jnp.zeros_like(gsum)
        gcnt[...] = jnp.zeros_like(gcnt)

    dis = dis_ref[...]
    s = s_ref[...]
    u = (s[0] + s[1]) * dis + b2_ref[...]
    z = jnp.where(u >= 0, u, 0.01 * u)
    gids = lax.broadcasted_iota(jnp.int32, (ROWBLK, NG), 1)
    m = (batch_ref[...] == gids).astype(jnp.float32)
    gsum[...] += lax.dot_general(m, z, (((0,), (0,)), ((), ())),
                                 precision=_HIGH)
    gcnt[...] += lax.dot_general(m, jnp.ones((ROWBLK, 1), jnp.float32),
                                 (((0,), (0,)), ((), ())), precision=_HIGH)

    @pl.when(i == NROWBLK - 1)
    def _():
        g = gsum[...] / jnp.maximum(gcnt[...], 1.0)
        out_ref[...] = lax.dot_general(g, wlin_ref[...],
                                       (((1,), (0,)), ((), ()))) + blin_ref[...]


@jax.jit
def _f_call(s2, dis, b2, batchp, wlin, blin):
    return pl.pallas_call(
        _f_body,
        grid=(NROWBLK,),
        in_specs=[
            pl.BlockSpec((NC, ROWBLK, H2F), lambda i: (0, i, 0)),
            pl.BlockSpec((ROWBLK, 1), lambda i: (i, 0)),
            pl.BlockSpec((1, H2F), lambda i: (0, 0)),
            pl.BlockSpec((ROWBLK, 1), lambda i: (i, 0)),
            pl.BlockSpec((H2F, NT), lambda i: (0, 0)),
            pl.BlockSpec((1, NT), lambda i: (0, 0)),
        ],
        out_specs=pl.BlockSpec((NG, NT), lambda i: (0, 0)),
        out_shape=jax.ShapeDtypeStruct((NG, NT), jnp.float32),
        scratch_shapes=[
            pltpu.VMEM((NG, H2F), jnp.float32),
            pltpu.VMEM((NG, 1), jnp.float32),
        ],
    )(s2, dis, b2, batchp, wlin, blin)


def kernel(x, edge_index, batch, W1, b1, W2, b2, Wlin, blin):
    src = edge_index[0]
    dst = edge_index[1]
    pad = jnp.full((EPAD - EDGES,), N, jnp.int32)
    srcp = jnp.concatenate([src, pad]).reshape(NCHTOT, CHUNK)
    dstp = jnp.concatenate([dst, pad]).reshape(NCHTOT, CHUNK)
    xp = jnp.pad(x, ((0, NPAD - N), (0, 0)))
    batchp = jnp.pad(batch, (0, NPAD - N), constant_values=NG).reshape(NPAD, 1)

    degp = _deg_call(dstp)
    dis, h1p = _b_call(degp, xp, W1)
    s1 = _prop64(h1p, srcp, dstp)
    h2p = _d_call(s1, dis, b1.reshape(1, H1F), W2)
    s2 = _prop32(h2p, srcp, dstp)
    return _f_call(s2, dis, b2.reshape(1, H2F), batchp,
                   Wlin, blin.reshape(1, NT))
